# Initial kernel scaffold; baseline (speedup 1.0000x reference)
import jax
import jax.numpy as jnp
from jax.experimental import pallas as pl


def kernel(inputs, table, W1, b1, W2, b2):
    raise NotImplementedError("write your pallas kernel here")


# speedup vs baseline: 46.1508x; 46.1508x over previous
import functools

import jax
import jax.numpy as jnp
from jax import lax
from jax.experimental import pallas as pl
from jax.experimental.pallas import tpu as pltpu
from jax.experimental.pallas import tpu_sc as plsc

VOCAB = 1000000
EMB = 3
B = 16384
L = 200
N = B * L


_ROWS = 8000


def _transform_body(w_ref, table_ref, out_ref):
    e = table_ref[...]
    w100 = w_ref[0]
    w101 = w_ref[1]
    w102 = w_ref[2]
    b10 = w_ref[3]
    w110 = w_ref[4]
    w111 = w_ref[5]
    w112 = w_ref[6]
    b11 = w_ref[7]
    w20 = w_ref[8]
    w21 = w_ref[9]
    b2 = w_ref[10]
    e0 = e[:, 0:1]
    e1 = e[:, 1:2]
    e2 = e[:, 2:3]
    h0 = jnp.maximum(e0 * w100 + e1 * w101 + e2 * w102 + b10, 0.0)
    h1 = jnp.maximum(e0 * w110 + e1 * w111 + e2 * w112 + b11, 0.0)
    out_ref[...] = jnp.maximum(h0 * w20 + h1 * w21 + b2, 0.0)


def _transform_table(table, W1, b1, W2, b2):
    w = jnp.concatenate(
        [W1[0], b1[0:1], W1[1], b1[1:2], W2[0], b2]
    ).astype(jnp.float32)
    grid = VOCAB // _ROWS
    out = pl.pallas_call(
        _transform_body,
        grid=(grid,),
        in_specs=[
            pl.BlockSpec(memory_space=pltpu.SMEM),
            pl.BlockSpec((_ROWS, EMB), lambda i: (i, 0)),
        ],
        out_specs=pl.BlockSpec((_ROWS, 1), lambda i: (i, 0)),
        out_shape=jax.ShapeDtypeStruct((VOCAB, 1), jnp.float32),
    )(w, table)
    return out.reshape(VOCAB)


_NC, _NS = 2, 16
_NW = _NC * _NS
_PER_W = N // _NW
_CHUNK = 12800
_NCHUNK = _PER_W // _CHUNK


def _gather_kernel(t_hbm, idx_hbm, out_hbm, idx_v, out_v, sem):
    wid = lax.axis_index("s") * _NC + lax.axis_index("c")
    base = wid * _PER_W

    def body(c, carry):
        off = base + c * _CHUNK
        pltpu.sync_copy(idx_hbm.at[pl.ds(off, _CHUNK)], idx_v)
        pltpu.async_copy(t_hbm.at[idx_v], out_v, sem).wait()
        pltpu.sync_copy(out_v, out_hbm.at[pl.ds(off, _CHUNK)])
        return carry

    lax.fori_loop(0, _NCHUNK, body, 0)


def _gather(t, idx_flat):
    mesh = plsc.VectorSubcoreMesh(core_axis_name="c", subcore_axis_name="s")
    run = functools.partial(
        pl.kernel,
        mesh=mesh,
        out_type=jax.ShapeDtypeStruct((N,), jnp.float32),
        scratch_types=[
            pltpu.VMEM((_CHUNK,), jnp.int32),
            pltpu.VMEM((_CHUNK,), jnp.float32),
            pltpu.SemaphoreType.DMA,
        ],
    )(_gather_kernel)
    return run(t, idx_flat)


def kernel(inputs, table, W1, b1, W2, b2):
    t = _transform_table(table, W1, b1, W2, b2)
    out = _gather(t, inputs.reshape(N))
    return out.reshape(B, L, 1)
